# baseline (speedup 1.0000x reference)
import functools

import jax
import jax.numpy as jnp
from jax import lax
from jax.experimental import pallas as pl
from jax.experimental.pallas import tpu as pltpu
from jax.experimental.pallas import tpu_sc as plsc

_NBUF = 4
_CHUNK = 400
_GROUP = 4


@functools.partial(jax.jit, static_argnums=(2, 3, 4))
def _embed_sc(xr, weight, nc, ns, n_fields):
    num, dim = weight.shape
    nw = nc * ns
    n_groups = xr.shape[1]
    gsz = _GROUP * n_fields
    bpw = _GROUP * n_groups
    bsz = nw * bpw
    n_chunks = num // _CHUNK
    chunks_per_sub = (n_chunks + ns - 1) // ns
    mesh = plsc.VectorSubcoreMesh(core_axis_name="c", subcore_axis_name="s")

    @functools.partial(
        pl.kernel,
        out_type=jax.ShapeDtypeStruct((bsz, n_fields, dim), jnp.float32),
        mesh=mesh,
        scratch_types=[
            pltpu.HBM((2, num, dim), jnp.float32),
            pltpu.VMEM((_CHUNK, dim), jnp.float32),
            pltpu.VMEM((n_groups, gsz), jnp.int32),
            pltpu.VMEM((_NBUF, gsz, dim), jnp.float32),
            pltpu.SemaphoreType.DMA((_NBUF,)),
            pltpu.SemaphoreType.DMA((_NBUF,)),
        ],
        compiler_params=pltpu.CompilerParams(use_tc_tiling_on_sc=False),
    )
    def k(x_hbm, w_hbm, out_hbm, tab_hbm, stage_v, idx_v, rows_v, gsem, wsem):
        cid = lax.axis_index("c")
        sid = lax.axis_index("s")
        wid = sid * nc + cid

        def copy_chunk(i, carry):
            c = sid + i * ns

            @pl.when(c < n_chunks)
            def _():
                rows = pl.ds(c * _CHUNK, _CHUNK)
                pltpu.sync_copy(w_hbm.at[rows], stage_v)
                pltpu.sync_copy(stage_v, tab_hbm.at[cid, rows])

            return carry

        lax.fori_loop(0, chunks_per_sub, copy_chunk, 0)
        plsc.subcore_barrier()

        pltpu.sync_copy(x_hbm.at[wid], idx_v)
        b0 = wid * bpw

        def start_gather(g, b):
            pltpu.make_async_copy(
                tab_hbm.at[cid].at[idx_v.at[g]], rows_v.at[b], gsem.at[b]
            ).start()

        def wait_gather(g, b):
            pltpu.make_async_copy(
                tab_hbm.at[cid].at[idx_v.at[g]], rows_v.at[b], gsem.at[b]
            ).wait()

        def write_copy(g, b, i):
            return pltpu.make_async_copy(
                rows_v.at[b, pl.ds(i * n_fields, n_fields)],
                out_hbm.at[b0 + g * _GROUP + i],
                wsem.at[b],
            )

        for b in range(_NBUF):
            start_gather(b, b)

        n_outer = (n_groups + _NBUF - 1) // _NBUF

        def body(j, carry):
            for b in range(_NBUF):
                g = j * _NBUF + b

                @pl.when(g < n_groups)
                def _():
                    wait_gather(g, b)
                    for i in range(_GROUP):
                        write_copy(g, b, i).start()

                    @pl.when(g + _NBUF < n_groups)
                    def _():
                        for i in range(_GROUP):
                            write_copy(g, b, i).wait()
                        start_gather(g + _NBUF, b)

            return carry

        lax.fori_loop(0, n_outer, body, 0)
        for b in range(_NBUF):
            for i in range(_GROUP):
                write_copy(0, b, i).wait()

    return k(xr, weight)


def kernel(x, weight):
    b, f = x.shape
    info = plsc.get_sparse_core_info()
    nc, ns = info.num_cores, info.num_subcores
    nw = nc * ns
    bpw = b // nw
    n_groups = bpw // _GROUP
    xr = x.reshape(nw, n_groups, _GROUP * f)
    return _embed_sc(xr, weight, nc, ns, f)


# speedup vs baseline: 1.2842x; 1.2842x over previous
import functools

import jax
import jax.numpy as jnp
from jax import lax
from jax.experimental import pallas as pl
from jax.experimental.pallas import tpu as pltpu
from jax.experimental.pallas import tpu_sc as plsc

_NBUF = 4


@functools.partial(jax.jit, static_argnums=(2, 3))
def _embed_sc(xt, weight, nc, ns):
    num, dim = weight.shape
    nw = nc * ns
    n_fields = xt.shape[1]
    bpw = xt.shape[2]
    bsz = nw * bpw
    mesh = plsc.VectorSubcoreMesh(core_axis_name="c", subcore_axis_name="s")

    @functools.partial(
        pl.kernel,
        out_type=jax.ShapeDtypeStruct((bsz, n_fields, dim), jnp.float32),
        mesh=mesh,
        scratch_types=[
            pltpu.VMEM((n_fields, bpw), jnp.int32),
            pltpu.VMEM((_NBUF, bpw, dim), jnp.float32),
            pltpu.SemaphoreType.DMA((_NBUF,)),
            pltpu.SemaphoreType.DMA((_NBUF,)),
        ],
        compiler_params=pltpu.CompilerParams(use_tc_tiling_on_sc=False),
    )
    def k(x_hbm, w_hbm, out_hbm, idx_v, rows_v, gsem, ssem):
        wid = lax.axis_index("s") * nc + lax.axis_index("c")
        b0 = wid * bpw
        pltpu.sync_copy(x_hbm.at[wid], idx_v)

        def start_gather(j, b):
            pltpu.make_async_copy(
                w_hbm.at[idx_v.at[j]], rows_v.at[b], gsem.at[b]
            ).start()

        def wait_gather(j, b):
            pltpu.make_async_copy(
                w_hbm.at[idx_v.at[j]], rows_v.at[b], gsem.at[b]
            ).wait()

        def store_copy(j, b):
            return pltpu.make_async_copy(
                rows_v.at[b],
                out_hbm.at[pl.ds(b0, bpw), j],
                ssem.at[b],
            )

        for b in range(_NBUF):
            start_gather(b, b)

        n_outer = (n_fields + _NBUF - 1) // _NBUF

        def body(j, carry):
            for b in range(_NBUF):
                c = j * _NBUF + b

                @pl.when(c < n_fields)
                def _():
                    wait_gather(c, b)
                    store_copy(c, b).start()

                    @pl.when(c + _NBUF < n_fields)
                    def _():
                        store_copy(c, b).wait()
                        start_gather(c + _NBUF, b)

            return carry

        lax.fori_loop(0, n_outer, body, 0)
        for b in range(_NBUF):
            store_copy(0, b).wait()

    return k(xt, weight)


def kernel(x, weight):
    b, f = x.shape
    info = plsc.get_sparse_core_info()
    nc, ns = info.num_cores, info.num_subcores
    nw = nc * ns
    bpw = b // nw
    xt = x.reshape(nw, bpw, f).transpose(0, 2, 1)
    return _embed_sc(xt, weight, nc, ns)


# speedup vs baseline: 1.7336x; 1.3500x over previous
import functools

import jax
import jax.numpy as jnp
from jax import lax
from jax.experimental import pallas as pl
from jax.experimental.pallas import tpu as pltpu
from jax.experimental.pallas import tpu_sc as plsc

_NBUF = 4
_GROUP = 4


@functools.partial(jax.jit, static_argnums=(2, 3, 4))
def _embed_sc(xflat, weight, nc, ns, n_fields):
    num, dim = weight.shape
    nw = nc * ns
    lpw = xflat.shape[1]
    gsz = _GROUP * n_fields
    n_groups = lpw // gsz
    bpw = lpw // n_fields
    bsz = nw * bpw
    mesh = plsc.VectorSubcoreMesh(core_axis_name="c", subcore_axis_name="s")

    @functools.partial(
        pl.kernel,
        out_type=jax.ShapeDtypeStruct((bsz, 32, 128), jnp.float32),
        mesh=mesh,
        scratch_types=[
            pltpu.VMEM((lpw,), jnp.int32),
            pltpu.VMEM((_NBUF, gsz, dim), jnp.float32),
            pltpu.SemaphoreType.DMA((_NBUF,)),
            pltpu.SemaphoreType.DMA((_NBUF,)),
        ],
        compiler_params=pltpu.CompilerParams(use_tc_tiling_on_sc=False),
    )
    def k(x_hbm, w_hbm, out_hbm, idx_v, rows_v, gsem, wsem):
        wid = lax.axis_index("s") * nc + lax.axis_index("c")
        b0 = wid * bpw
        pltpu.sync_copy(x_hbm.at[wid], idx_v)

        def start_gather(g, b):
            pltpu.make_async_copy(
                w_hbm.at[idx_v.at[pl.ds(g * gsz, gsz)]], rows_v.at[b], gsem.at[b]
            ).start()

        def wait_gather(g, b):
            pltpu.make_async_copy(
                w_hbm.at[idx_v.at[pl.ds(g * gsz, gsz)]], rows_v.at[b], gsem.at[b]
            ).wait()

        def write_copy(g, b, i):
            return pltpu.make_async_copy(
                rows_v.at[b, pl.ds(i * n_fields, n_fields)],
                out_hbm.at[b0 + g * _GROUP + i, pl.ds(0, n_fields), pl.ds(0, dim)],
                wsem.at[b],
            )

        for b in range(_NBUF):
            start_gather(b, b)

        n_outer = (n_groups + _NBUF - 1) // _NBUF

        def body(j, carry):
            for b in range(_NBUF):
                g = j * _NBUF + b

                @pl.when(g < n_groups)
                def _():
                    wait_gather(g, b)
                    for i in range(_GROUP):
                        write_copy(g, b, i).start()

                    @pl.when(g + _NBUF < n_groups)
                    def _():
                        for i in range(_GROUP):
                            write_copy(g, b, i).wait()
                        start_gather(g + _NBUF, b)

            return carry

        lax.fori_loop(0, n_outer, body, 0)
        for b in range(_NBUF):
            for i in range(_GROUP):
                write_copy(0, b, i).wait()

    out_big = k(xflat, weight)
    return out_big[:, :n_fields, :dim]


def kernel(x, weight):
    b, f = x.shape
    info = plsc.get_sparse_core_info()
    nc, ns = info.num_cores, info.num_subcores
    nw = nc * ns
    xflat = x.reshape(nw, (b // nw) * f)
    return _embed_sc(xflat, weight, nc, ns, f)
